# baseline (speedup 1.0000x reference)
import functools

import jax
import jax.numpy as jnp
from jax.experimental import pallas as pl
from jax.experimental.pallas import tpu as pltpu

_N = 10000
_G = 64
_H = 4
_C = 256


def _mm_kernel(a_ref, b_ref, o_ref):
    k = pl.program_id(2)

    @pl.when(k == 0)
    def _():
        o_ref[...] = jnp.zeros_like(o_ref)

    o_ref[...] += jnp.dot(a_ref[...], b_ref[...],
                          preferred_element_type=jnp.float32)


def _matmul(a, b, bm=512, bn=512, bk=512):
    m, k = a.shape
    k2, n = b.shape
    assert k == k2
    mp = (m + bm - 1) // bm * bm
    kp = (k + bk - 1) // bk * bk
    np_ = (n + bn - 1) // bn * bn
    if mp != m or kp != k:
        a = jnp.pad(a, ((0, mp - m), (0, kp - k)))
    if kp != k or np_ != n:
        b = jnp.pad(b, ((0, kp - k), (0, np_ - n)))
    out = pl.pallas_call(
        _mm_kernel,
        grid=(mp // bm, np_ // bn, kp // bk),
        in_specs=[
            pl.BlockSpec((bm, bk), lambda i, j, k: (i, k)),
            pl.BlockSpec((bk, bn), lambda i, j, k: (k, j)),
        ],
        out_specs=pl.BlockSpec((bm, bn), lambda i, j, k: (i, j)),
        out_shape=jax.ShapeDtypeStruct((mp, np_), jnp.float32),
    )(a, b)
    return out[:m, :n]


def _segment_softmax(scores, seg, num_segments):
    m = jax.ops.segment_max(scores, seg, num_segments=num_segments)
    m = jnp.where(jnp.isfinite(m), m, 0.0)
    e = jnp.exp(scores - m[seg])
    s = jax.ops.segment_sum(e, seg, num_segments=num_segments)
    return e / (s[seg] + 1e-16)


def _gatv2(x, edge_index, edge_attr, Wl, bl, Wr, br, We, att, bias, concat):
    n = x.shape[0]
    src0 = edge_index[0]
    dst0 = edge_index[1]
    deg = jax.ops.segment_sum(jnp.ones((src0.shape[0],), jnp.float32), dst0,
                              num_segments=n)
    self_attr = jax.ops.segment_sum(edge_attr, dst0, num_segments=n) \
        / jnp.maximum(deg, 1.0)[:, None]
    loop = jnp.arange(n, dtype=src0.dtype)
    src = jnp.concatenate([src0, loop])
    dst = jnp.concatenate([dst0, loop])
    ea = jnp.concatenate([edge_attr, self_attr], axis=0)
    xlr = _matmul(x, jnp.concatenate([Wl, Wr], axis=1)) \
        + jnp.concatenate([bl, br])
    xl = xlr[:, : _H * _C].reshape(n, _H, _C)
    xr = xlr[:, _H * _C:].reshape(n, _H, _C)
    ee = (ea @ We).reshape(-1, _H, _C)
    m = xl[src] + xr[dst] + ee
    m = jax.nn.leaky_relu(m, 0.2)
    alpha = jnp.sum(m * att[None, :, :], axis=-1)
    alpha = _segment_softmax(alpha, dst, n)
    msg = xl[src] * alpha[:, :, None]
    out = jax.ops.segment_sum(msg, dst, num_segments=n)
    if concat:
        out = out.reshape(n, _H * _C)
    else:
        out = out.mean(axis=1)
    return out + bias


def kernel(x, edge_index, edge_attr, batch, W_l1, b_l1, W_r1, b_r1, W_e1,
           att1, bias1, W_l2, b_l2, W_r2, b_r2, W_e2, att2, bias2, conv1_w,
           conv1_b, conv2_w, conv2_b, fc_seq_w, fc_seq_b, fc_w, fc_b):
    h = jax.nn.relu(_gatv2(x, edge_index, edge_attr, W_l1, b_l1, W_r1, b_r1,
                           W_e1, att1, bias1, True))
    h = jax.nn.relu(_gatv2(h, edge_index, edge_attr, W_l2, b_l2, W_r2, b_r2,
                           W_e2, att2, bias2, False))

    counts = jax.ops.segment_sum(jnp.ones((_N,), jnp.float32), batch,
                                 num_segments=_G)
    denom = jnp.maximum(counts, 1.0)[:, None]
    mean_pool = jax.ops.segment_sum(h, batch, num_segments=_G) / denom
    max_pool = jax.ops.segment_max(h, batch, num_segments=_G)

    hm1 = jnp.pad(h[:, :-1], ((0, 0), (1, 0)))
    hp1 = jnp.pad(h[:, 1:], ((0, 0), (0, 1)))
    w1 = conv1_w[:, 0, :]
    y1 = (hm1[:, :, None] * w1[None, None, :, 0]
          + h[:, :, None] * w1[None, None, :, 1]
          + hp1[:, :, None] * w1[None, None, :, 2]) + conv1_b[None, None, :]
    y1 = jax.nn.relu(y1)
    p1 = y1.reshape(_N, 128, 2, 32).max(axis=2)

    pm1 = jnp.pad(p1[:, :-1], ((0, 0), (1, 0), (0, 0)))
    pp1 = jnp.pad(p1[:, 1:], ((0, 0), (0, 1), (0, 0)))
    w2 = conv2_w
    patches = jnp.concatenate([pm1, p1, pp1], axis=-1).reshape(_N * 128, 96)
    w2cat = jnp.concatenate([w2[:, :, 0], w2[:, :, 1], w2[:, :, 2]],
                            axis=1).T
    y2 = _matmul(patches, w2cat).reshape(_N, 128, 64) \
        + conv2_b[None, None, :]
    y2 = jax.nn.relu(y2)
    p2 = y2.reshape(_N, 64, 2, 64).max(axis=2)
    flat = p2.transpose(0, 2, 1).reshape(_N, 64 * 64)

    mean_flat = jax.ops.segment_sum(flat, batch, num_segments=_G) / denom
    bscale = (counts / jnp.maximum(counts, 1.0))[:, None]
    hs_graph = mean_flat @ fc_seq_w + bscale * fc_seq_b[None, :]

    gf = jnp.concatenate([mean_pool, max_pool, hs_graph], axis=-1)
    return gf @ fc_w + fc_b
